# Initial kernel scaffold; baseline (speedup 1.0000x reference)
import jax
import jax.numpy as jnp
from jax.experimental import pallas as pl


def kernel(xyz, new_xyz, features, W1, b1, W2, b2):
    raise NotImplementedError("write your pallas kernel here")


# speedup vs baseline: 8.9751x; 8.9751x over previous
import functools

import jax
import jax.numpy as jnp
from jax import lax
from jax.experimental import pallas as pl
from jax.experimental.pallas import tpu as pltpu
from jax.experimental.pallas import tpu_sc as plsc

B, N, NP, NS, C = 8, 4096, 1024, 32, 64
RADIUS = 0.2
C_OUT1, C_OUT2 = 64, 128

NC, NSUB = 2, 16
NW = NC * NSUB
SEGS = NW // B
P_PER_W = NP // SEGS
ROWS = B * NP * NS
ROWS_PER_W = ROWS // NW
GCHUNK = 128
KPAD = 128


R_BLK = 512
P_BLK = R_BLK // NS


def _sc_body(u_hbm, xyzt_hbm, cxyzt_hbm, gout_hbm,
             xw, yw, zw, cxw, cyw, czw, rowbuf, idxflat, gbuf, sem):
    wid = lax.axis_index("s") * NC + lax.axis_index("c")
    b = wid // SEGS
    seg = wid % SEGS
    boff = b * N

    pltpu.sync_copy(xyzt_hbm.at[b * 3 + 0], xw)
    pltpu.sync_copy(xyzt_hbm.at[b * 3 + 1], yw)
    pltpu.sync_copy(xyzt_hbm.at[b * 3 + 2], zw)
    pltpu.sync_copy(cxyzt_hbm.at[b * 3 + 0, pl.ds(seg * P_PER_W, P_PER_W)],
                    cxw.at[pl.ds(0, P_PER_W)])
    pltpu.sync_copy(cxyzt_hbm.at[b * 3 + 1, pl.ds(seg * P_PER_W, P_PER_W)],
                    cyw.at[pl.ds(0, P_PER_W)])
    pltpu.sync_copy(cxyzt_hbm.at[b * 3 + 2, pl.ds(seg * P_PER_W, P_PER_W)],
                    czw.at[pl.ds(0, P_PER_W)])

    r2 = jnp.float32(RADIUS * RADIUS)
    lane = lax.iota(jnp.int32, 16)

    def per_centroid(p, _):
        cx = cxw[pl.ds(p, 16)][0]
        cy = cyw[pl.ds(p, 16)][0]
        cz = czw[pl.ds(p, 16)][0]

        def chunk(jj, cv):
            xv = xw[pl.ds(jj, 16)]
            yv = yw[pl.ds(jj, 16)]
            zv = zw[pl.ds(jj, 16)]
            dx = xv - cx
            dy = yv - cy
            dz = zv - cz
            d = dx * dx + dy * dy
            d = d + dz * dz
            m = d <= r2
            csum = plsc.cumsum(m.astype(jnp.int32))
            slot = cv + csum - 1
            meff = jnp.logical_and(m, slot < NS)
            iv = jj + lane
            plsc.store_scatter(rowbuf, [slot], iv, mask=meff)
            return cv + plsc.all_reduce_population_count(m)

        cntv = lax.fori_loop(0, N // 16, lambda j, cv: chunk(j * 16, cv),
                             jnp.zeros((16,), jnp.int32))

        firstv = plsc.load_gather(rowbuf, [jnp.zeros((16,), jnp.int32)])
        firstv = jnp.where(cntv > 0, firstv, jnp.int32(N - 1))
        base = p * NS
        for h in range(NS // 16):
            v = rowbuf[pl.ds(h * 16, 16)]
            sidv = h * 16 + lane
            vv = jnp.where(sidv < cntv, v, firstv)
            idxflat[pl.ds(base + h * 16, 16)] = vv + boff
        return _

    lax.fori_loop(0, P_PER_W, per_centroid, 0)

    def per_chunk(k, _):
        idx_sl = idxflat.at[pl.ds(k * GCHUNK, GCHUNK)]
        pltpu.async_copy(u_hbm.at[idx_sl], gbuf, sem).wait()
        pltpu.sync_copy(gbuf, gout_hbm.at[pl.ds(wid * ROWS_PER_W + k * GCHUNK,
                                                GCHUNK)])
        return _

    lax.fori_loop(0, ROWS_PER_W // GCHUNK, per_chunk, 0)


@functools.partial(
    pl.kernel,
    out_type=jax.ShapeDtypeStruct((ROWS, KPAD), jnp.float32),
    mesh=plsc.VectorSubcoreMesh(core_axis_name="c", subcore_axis_name="s"),
    compiler_params=pltpu.CompilerParams(needs_layout_passes=False),
    scratch_types=[
        pltpu.VMEM((N,), jnp.float32),
        pltpu.VMEM((N,), jnp.float32),
        pltpu.VMEM((N,), jnp.float32),
        pltpu.VMEM((P_PER_W + 16,), jnp.float32),
        pltpu.VMEM((P_PER_W + 16,), jnp.float32),
        pltpu.VMEM((P_PER_W + 16,), jnp.float32),
        pltpu.VMEM((NS,), jnp.int32),
        pltpu.VMEM((ROWS_PER_W,), jnp.int32),
        pltpu.VMEM((GCHUNK, KPAD), jnp.float32),
        pltpu.SemaphoreType.DMA,
    ],
)
def _sc_select_gather(u_hbm, xyzt_hbm, cxyzt_hbm, gout_hbm, *scratch):
    _sc_body(u_hbm, xyzt_hbm, cxyzt_hbm, gout_hbm, *scratch)


def _mlp_body(g_ref, c_ref, w1_ref, w2_ref, b1_ref, b2_ref, o_ref):
    g = g_ref[...]
    w1 = w1_ref[...]
    h = jnp.dot(g, w1, preferred_element_type=jnp.float32)
    cp = c_ref[...]
    w1x = w1[C:C + 3, :]
    cpw = jnp.dot(cp, w1x, preferred_element_type=jnp.float32)
    cpr = jnp.broadcast_to(cpw[:, None, :], (P_BLK, NS, C_OUT1))
    cpr = cpr.reshape(R_BLK, C_OUT1)
    h1 = jnp.maximum(h - cpr + b1_ref[...], 0.0)
    h2 = jnp.dot(h1, w2_ref[...], preferred_element_type=jnp.float32)
    h2 = jnp.maximum(h2 + b2_ref[...], 0.0)
    o_ref[...] = jnp.max(h2.reshape(P_BLK, NS, C_OUT2), axis=1)


def _tc_mlp(gu, nflat, w1pt, w2t, b1, b2):
    grid = (ROWS // R_BLK,)
    return pl.pallas_call(
        _mlp_body,
        grid=grid,
        in_specs=[
            pl.BlockSpec((R_BLK, KPAD), lambda i: (i, 0)),
            pl.BlockSpec((P_BLK, 3), lambda i: (i, 0)),
            pl.BlockSpec((KPAD, C_OUT1), lambda i: (0, 0)),
            pl.BlockSpec((C_OUT1, C_OUT2), lambda i: (0, 0)),
            pl.BlockSpec((1, C_OUT1), lambda i: (0, 0)),
            pl.BlockSpec((1, C_OUT2), lambda i: (0, 0)),
        ],
        out_specs=pl.BlockSpec((P_BLK, C_OUT2), lambda i: (i, 0)),
        out_shape=jax.ShapeDtypeStruct((B * NP, C_OUT2), jnp.float32),
    )(gu, nflat, w1pt, w2t, b1, b2)


@jax.jit
def kernel(xyz, new_xyz, features, W1, b1, W2, b2):
    feat_t = jnp.transpose(features, (0, 2, 1))
    u = jnp.concatenate(
        [feat_t, xyz, jnp.zeros((B, N, KPAD - C - 3), jnp.float32)], axis=-1)
    u = u.reshape(B * N, KPAD)
    xyzt = jnp.transpose(xyz, (0, 2, 1)).reshape(B * 3, N)
    cxyzt = jnp.transpose(new_xyz, (0, 2, 1)).reshape(B * 3, NP)

    gu = _sc_select_gather(u, xyzt, cxyzt)

    w1pt = jnp.concatenate(
        [W1[:, 3:].T, W1[:, :3].T, jnp.zeros((KPAD - C - 3, C_OUT1),
                                             jnp.float32)], axis=0)
    w2t = W2.T
    nflat = new_xyz.reshape(B * NP, 3)
    out = _tc_mlp(gu, nflat, w1pt, w2t, b1[None, :], b2[None, :])

    new_features = out.reshape(B, NP, C_OUT2).transpose(0, 2, 1)
    return (new_xyz, new_features)


# speedup vs baseline: 20.6018x; 2.2954x over previous
import functools

import jax
import jax.numpy as jnp
from jax import lax
from jax.experimental import pallas as pl
from jax.experimental.pallas import tpu as pltpu
from jax.experimental.pallas import tpu_sc as plsc

B, N, NP, NS, C = 8, 4096, 1024, 32, 64
RADIUS = 0.2
C_OUT1, C_OUT2 = 64, 128

NC, NSUB = 2, 16
NW = NC * NSUB
SEGS = NW // B
P_PER_W = NP // SEGS
ROWS = B * NP * NS
ROWS_PER_W = ROWS // NW
GCHUNK = 128
KPAD = 128


R_BLK = 512
P_BLK = R_BLK // NS


def _sc_body(u_hbm, xyzt_hbm, cxyzt_hbm, gout_hbm,
             xw, yw, zw, cxw, cyw, czw, rowbuf, idxflat, gbuf, sem):
    wid = lax.axis_index("s") * NC + lax.axis_index("c")
    b = wid // SEGS
    seg = wid % SEGS
    boff = b * N

    pltpu.sync_copy(xyzt_hbm.at[b * 3 + 0], xw)
    pltpu.sync_copy(xyzt_hbm.at[b * 3 + 1], yw)
    pltpu.sync_copy(xyzt_hbm.at[b * 3 + 2], zw)
    pltpu.sync_copy(cxyzt_hbm.at[b * 3 + 0, pl.ds(seg * P_PER_W, P_PER_W)],
                    cxw.at[pl.ds(0, P_PER_W)])
    pltpu.sync_copy(cxyzt_hbm.at[b * 3 + 1, pl.ds(seg * P_PER_W, P_PER_W)],
                    cyw.at[pl.ds(0, P_PER_W)])
    pltpu.sync_copy(cxyzt_hbm.at[b * 3 + 2, pl.ds(seg * P_PER_W, P_PER_W)],
                    czw.at[pl.ds(0, P_PER_W)])

    r2 = jnp.float32(RADIUS * RADIUS)
    lane = lax.iota(jnp.int32, 16)

    def per_centroid(p, _):
        cx = cxw[pl.ds(p, 16)][0]
        cy = cyw[pl.ds(p, 16)][0]
        cz = czw[pl.ds(p, 16)][0]

        def chunk(jj, cv):
            xv = xw[pl.ds(jj, 16)]
            yv = yw[pl.ds(jj, 16)]
            zv = zw[pl.ds(jj, 16)]
            dx = xv - cx
            dy = yv - cy
            dz = zv - cz
            d = dx * dx + dy * dy
            d = d + dz * dz
            m = d <= r2
            csum = plsc.cumsum(m.astype(jnp.int32))
            slot = cv + csum - 1
            meff = jnp.logical_and(m, slot < NS)
            iv = jj + lane
            plsc.store_scatter(rowbuf, [slot], iv, mask=meff)
            return cv + plsc.all_reduce_population_count(m)

        cntv = plsc.parallel_loop(0, N, 16, unroll=4,
                                  carry=jnp.zeros((16,), jnp.int32))(chunk)

        firstv = plsc.load_gather(rowbuf, [jnp.zeros((16,), jnp.int32)])
        firstv = jnp.where(cntv > 0, firstv, jnp.int32(N - 1))
        base = p * NS
        for h in range(NS // 16):
            v = rowbuf[pl.ds(h * 16, 16)]
            sidv = h * 16 + lane
            vv = jnp.where(sidv < cntv, v, firstv)
            idxflat[pl.ds(base + h * 16, 16)] = vv + boff
        return _

    lax.fori_loop(0, P_PER_W, per_centroid, 0)

    def per_chunk(k, _):
        idx_sl = idxflat.at[pl.ds(k * GCHUNK, GCHUNK)]
        pltpu.async_copy(u_hbm.at[idx_sl], gbuf, sem).wait()
        pltpu.sync_copy(gbuf, gout_hbm.at[pl.ds(wid * ROWS_PER_W + k * GCHUNK,
                                                GCHUNK)])
        return _

    lax.fori_loop(0, ROWS_PER_W // GCHUNK, per_chunk, 0)


@functools.partial(
    pl.kernel,
    out_type=jax.ShapeDtypeStruct((ROWS, KPAD), jnp.float32),
    mesh=plsc.VectorSubcoreMesh(core_axis_name="c", subcore_axis_name="s"),
    compiler_params=pltpu.CompilerParams(needs_layout_passes=False),
    scratch_types=[
        pltpu.VMEM((N,), jnp.float32),
        pltpu.VMEM((N,), jnp.float32),
        pltpu.VMEM((N,), jnp.float32),
        pltpu.VMEM((P_PER_W + 16,), jnp.float32),
        pltpu.VMEM((P_PER_W + 16,), jnp.float32),
        pltpu.VMEM((P_PER_W + 16,), jnp.float32),
        pltpu.VMEM((NS,), jnp.int32),
        pltpu.VMEM((ROWS_PER_W,), jnp.int32),
        pltpu.VMEM((GCHUNK, KPAD), jnp.float32),
        pltpu.SemaphoreType.DMA,
    ],
)
def _sc_select_gather(u_hbm, xyzt_hbm, cxyzt_hbm, gout_hbm, *scratch):
    _sc_body(u_hbm, xyzt_hbm, cxyzt_hbm, gout_hbm, *scratch)


def _mlp_body(g_ref, c_ref, w1_ref, w2_ref, b1_ref, b2_ref, o_ref):
    g = g_ref[...]
    w1 = w1_ref[...]
    h = jnp.dot(g, w1, preferred_element_type=jnp.float32)
    cp = c_ref[...]
    w1x = w1[C:C + 3, :]
    cpw = jnp.dot(cp, w1x, preferred_element_type=jnp.float32)
    cpr = jnp.broadcast_to(cpw[:, None, :], (P_BLK, NS, C_OUT1))
    cpr = cpr.reshape(R_BLK, C_OUT1)
    h1 = jnp.maximum(h - cpr + b1_ref[...], 0.0)
    h2 = jnp.dot(h1, w2_ref[...], preferred_element_type=jnp.float32)
    h2 = jnp.maximum(h2 + b2_ref[...], 0.0)
    o_ref[...] = jnp.max(h2.reshape(P_BLK, NS, C_OUT2), axis=1)


def _tc_mlp(gu, nflat, w1pt, w2t, b1, b2):
    grid = (ROWS // R_BLK,)
    return pl.pallas_call(
        _mlp_body,
        grid=grid,
        in_specs=[
            pl.BlockSpec((R_BLK, KPAD), lambda i: (i, 0)),
            pl.BlockSpec((P_BLK, 3), lambda i: (i, 0)),
            pl.BlockSpec((KPAD, C_OUT1), lambda i: (0, 0)),
            pl.BlockSpec((C_OUT1, C_OUT2), lambda i: (0, 0)),
            pl.BlockSpec((1, C_OUT1), lambda i: (0, 0)),
            pl.BlockSpec((1, C_OUT2), lambda i: (0, 0)),
        ],
        out_specs=pl.BlockSpec((P_BLK, C_OUT2), lambda i: (i, 0)),
        out_shape=jax.ShapeDtypeStruct((B * NP, C_OUT2), jnp.float32),
    )(gu, nflat, w1pt, w2t, b1, b2)


@jax.jit
def kernel(xyz, new_xyz, features, W1, b1, W2, b2):
    feat_t = jnp.transpose(features, (0, 2, 1))
    u = jnp.concatenate(
        [feat_t, xyz, jnp.zeros((B, N, KPAD - C - 3), jnp.float32)], axis=-1)
    u = u.reshape(B * N, KPAD)
    xyzt = jnp.transpose(xyz, (0, 2, 1)).reshape(B * 3, N)
    cxyzt = jnp.transpose(new_xyz, (0, 2, 1)).reshape(B * 3, NP)

    gu = _sc_select_gather(u, xyzt, cxyzt)

    w1pt = jnp.concatenate(
        [W1[:, 3:].T, W1[:, :3].T, jnp.zeros((KPAD - C - 3, C_OUT1),
                                             jnp.float32)], axis=0)
    w2t = W2.T
    nflat = new_xyz.reshape(B * NP, 3)
    out = _tc_mlp(gu, nflat, w1pt, w2t, b1[None, :], b2[None, :])

    new_features = out.reshape(B, NP, C_OUT2).transpose(0, 2, 1)
    return (new_xyz, new_features)


# speedup vs baseline: 42.5292x; 2.0643x over previous
import functools

import jax
import jax.numpy as jnp
from jax import lax
from jax.experimental import pallas as pl
from jax.experimental.pallas import tpu as pltpu
from jax.experimental.pallas import tpu_sc as plsc

B, N, NP, NS, C = 8, 4096, 1024, 32, 64
RADIUS = 0.2
C_OUT1, C_OUT2 = 64, 128

NC, NSUB = 2, 16
NW = NC * NSUB
ROWS = B * NP * NS
GCHUNK = 128
KPAD = 128

R_BLK = 2048
P_BLK = R_BLK // NS

BH = 4


def _make_sc(b0):
    segs = NW // BH
    ppw = NP // segs
    rows_l = BH * NP * NS
    rpw = rows_l // NW
    ng = rpw // GCHUNK
    gpc = ppw // ng

    def body(u_hbm, xyzt_hbm, cxyzt_hbm, gout_hbm,
             xw, yw, zw, cxw, cyw, czw, rowbuf, idxflat,
             gbuf0, gbuf1, gsem0, gsem1, csem0, csem1):
        wid = lax.axis_index("s") * NC + lax.axis_index("c")
        b = wid // segs + b0
        seg = wid % segs
        boff = b * N

        pltpu.sync_copy(xyzt_hbm.at[b * 3 + 0], xw)
        pltpu.sync_copy(xyzt_hbm.at[b * 3 + 1], yw)
        pltpu.sync_copy(xyzt_hbm.at[b * 3 + 2], zw)
        pltpu.sync_copy(cxyzt_hbm.at[b * 3 + 0, pl.ds(seg * ppw, ppw)],
                        cxw.at[pl.ds(0, ppw)])
        pltpu.sync_copy(cxyzt_hbm.at[b * 3 + 1, pl.ds(seg * ppw, ppw)],
                        cyw.at[pl.ds(0, ppw)])
        pltpu.sync_copy(cxyzt_hbm.at[b * 3 + 2, pl.ds(seg * ppw, ppw)],
                        czw.at[pl.ds(0, ppw)])

        r2 = jnp.float32(RADIUS * RADIUS)
        lane = lax.iota(jnp.int32, 16)

        def per_centroid(p):
            cx = cxw[pl.ds(p, 16)][0]
            cy = cyw[pl.ds(p, 16)][0]
            cz = czw[pl.ds(p, 16)][0]

            def chunk(jj, cv):
                xv = xw[pl.ds(jj, 16)]
                yv = yw[pl.ds(jj, 16)]
                zv = zw[pl.ds(jj, 16)]
                dx = xv - cx
                dy = yv - cy
                dz = zv - cz
                d = dx * dx + dy * dy
                d = d + dz * dz
                m = d <= r2
                csum = plsc.cumsum(m.astype(jnp.int32))
                slot = cv + csum - 1
                meff = jnp.logical_and(m, slot < NS)
                iv = jj + lane
                plsc.store_scatter(rowbuf, [slot], iv, mask=meff)
                return cv + plsc.all_reduce_population_count(m)

            cntv = plsc.parallel_loop(0, N, 16, unroll=4,
                                      carry=jnp.zeros((16,), jnp.int32))(chunk)

            firstv = plsc.load_gather(rowbuf, [jnp.zeros((16,), jnp.int32)])
            firstv = jnp.where(cntv > 0, firstv, jnp.int32(N - 1))
            base = p * NS
            for h in range(NS // 16):
                v = rowbuf[pl.ds(h * 16, 16)]
                sidv = h * 16 + lane
                vv = jnp.where(sidv < cntv, v, firstv)
                idxflat[pl.ds(base + h * 16, 16)] = vv + boff

        def sel(g):
            for u in range(gpc):
                per_centroid(g * gpc + u)

        def idx_sl(g):
            return idxflat.at[pl.ds(g * GCHUNK, GCHUNK)]

        def out_sl(g):
            return gout_hbm.at[pl.ds(wid * rpw + g * GCHUNK, GCHUNK)]

        def fire_gather(g, gb, gs):
            pltpu.async_copy(u_hbm.at[idx_sl(g)], gb, gs)

        def wait_gather(g, gb, gs):
            pltpu.make_async_copy(u_hbm.at[idx_sl(g)], gb, gs).wait()

        def fire_copy(g, gb, cs):
            pltpu.async_copy(gb, out_sl(g), cs)

        def wait_copy(g, gb, cs):
            pltpu.make_async_copy(gb, out_sl(g), cs).wait()

        sel(0)
        fire_gather(0, gbuf0, gsem0)
        sel(1)
        wait_gather(0, gbuf0, gsem0)
        fire_copy(0, gbuf0, csem0)
        fire_gather(1, gbuf1, gsem1)

        def pipe(t, _):
            g0 = 2 * t
            g1 = 2 * t + 1
            sel(g0)
            wait_gather(g1 - 2, gbuf1, gsem1)
            fire_copy(g1 - 2, gbuf1, csem1)
            wait_copy(g0 - 2, gbuf0, csem0)
            fire_gather(g0, gbuf0, gsem0)
            sel(g1)
            wait_gather(g0, gbuf0, gsem0)
            fire_copy(g0, gbuf0, csem0)
            wait_copy(g1 - 2, gbuf1, csem1)
            fire_gather(g1, gbuf1, gsem1)
            return _

        lax.fori_loop(1, ng // 2, pipe, 0)

        wait_gather(ng - 1, gbuf1, gsem1)
        fire_copy(ng - 1, gbuf1, csem1)
        wait_copy(ng - 2, gbuf0, csem0)
        wait_copy(ng - 1, gbuf1, csem1)

    return pl.kernel(
        body,
        out_type=jax.ShapeDtypeStruct((rows_l, KPAD), jnp.float32),
        mesh=plsc.VectorSubcoreMesh(core_axis_name="c", subcore_axis_name="s"),
        compiler_params=pltpu.CompilerParams(needs_layout_passes=False),
        scratch_types=[
            pltpu.VMEM((N,), jnp.float32),
            pltpu.VMEM((N,), jnp.float32),
            pltpu.VMEM((N,), jnp.float32),
            pltpu.VMEM((ppw + 16,), jnp.float32),
            pltpu.VMEM((ppw + 16,), jnp.float32),
            pltpu.VMEM((ppw + 16,), jnp.float32),
            pltpu.VMEM((NS,), jnp.int32),
            pltpu.VMEM((rpw,), jnp.int32),
            pltpu.VMEM((GCHUNK, KPAD), jnp.float32),
            pltpu.VMEM((GCHUNK, KPAD), jnp.float32),
            pltpu.SemaphoreType.DMA,
            pltpu.SemaphoreType.DMA,
            pltpu.SemaphoreType.DMA,
            pltpu.SemaphoreType.DMA,
        ],
    )


_sc_half0 = _make_sc(0)
_sc_half1 = _make_sc(BH)


def _mlp_body(g_ref, c_ref, w1_ref, w2_ref, b1_ref, b2_ref, o_ref):
    g = g_ref[...]
    w1 = w1_ref[...]
    h = jnp.dot(g, w1, preferred_element_type=jnp.float32)
    cp = c_ref[...]
    w1x = w1[C:C + 3, :]
    cpw = jnp.dot(cp, w1x, preferred_element_type=jnp.float32)
    cpr = jnp.broadcast_to(cpw[:, None, :], (P_BLK, NS, C_OUT1))
    cpr = cpr.reshape(R_BLK, C_OUT1)
    h1 = jnp.maximum(h - cpr + b1_ref[...], 0.0)
    h2 = jnp.dot(h1, w2_ref[...], preferred_element_type=jnp.float32)
    h2 = jnp.maximum(h2 + b2_ref[...], 0.0)
    o_ref[...] = jnp.max(h2.reshape(P_BLK, NS, C_OUT2), axis=1)


def _tc_mlp(gu, nflat, w1pt, w2t, b1, b2):
    rows_l = gu.shape[0]
    grid = (rows_l // R_BLK,)
    return pl.pallas_call(
        _mlp_body,
        grid=grid,
        in_specs=[
            pl.BlockSpec((R_BLK, KPAD), lambda i: (i, 0)),
            pl.BlockSpec((P_BLK, 3), lambda i: (i, 0)),
            pl.BlockSpec((KPAD, C_OUT1), lambda i: (0, 0)),
            pl.BlockSpec((C_OUT1, C_OUT2), lambda i: (0, 0)),
            pl.BlockSpec((1, C_OUT1), lambda i: (0, 0)),
            pl.BlockSpec((1, C_OUT2), lambda i: (0, 0)),
        ],
        out_specs=pl.BlockSpec((P_BLK, C_OUT2), lambda i: (i, 0)),
        out_shape=jax.ShapeDtypeStruct((rows_l // NS, C_OUT2), jnp.float32),
    )(gu, nflat, w1pt, w2t, b1, b2)


@jax.jit
def kernel(xyz, new_xyz, features, W1, b1, W2, b2):
    feat_t = jnp.transpose(features, (0, 2, 1))
    u = jnp.concatenate(
        [feat_t, xyz, jnp.zeros((B, N, KPAD - C - 3), jnp.float32)], axis=-1)
    u = u.reshape(B * N, KPAD)
    xyzt = jnp.transpose(xyz, (0, 2, 1)).reshape(B * 3, N)
    cxyzt = jnp.transpose(new_xyz, (0, 2, 1)).reshape(B * 3, NP)

    w1pt = jnp.concatenate(
        [W1[:, 3:].T, W1[:, :3].T, jnp.zeros((KPAD - C - 3, C_OUT1),
                                             jnp.float32)], axis=0)
    w2t = W2.T
    nflat = new_xyz.reshape(B * NP, 3)
    half = BH * NP

    gu0 = _sc_half0(u, xyzt, cxyzt)
    gu1 = _sc_half1(u, xyzt, cxyzt)
    out0 = _tc_mlp(gu0, nflat[:half], w1pt, w2t, b1[None, :], b2[None, :])
    out1 = _tc_mlp(gu1, nflat[half:], w1pt, w2t, b1[None, :], b2[None, :])

    out = jnp.concatenate([out0, out1], axis=0)
    new_features = out.reshape(B, NP, C_OUT2).transpose(0, 2, 1)
    return (new_xyz, new_features)
